# baseline (speedup 1.0000x reference)
import functools

import jax
import jax.numpy as jnp
from jax import lax
from jax.experimental import pallas as pl
from jax.experimental.pallas import tpu as pltpu
from jax.experimental.pallas import tpu_sc as plsc

N = 10000
E = 160000
HD = 128
ED = 300
LANES = 16

EPAD = 163840
EPW = EPAD // 16
KB = 64
NBLK = EPW // KB
NP = 10240
DUMMY = 10000
HALF = NP // 2
NPH = 6144
TRASH = HALF
NPC = NP // 8
CHALF = NPC // 2
NPCH = 768
TRASHC = CHALF
MROWS_PT = HALF // 16
CROWS_PT = CHALF // 16
ZBW = 2048


def _mm_body(x_ref, w_ref, o_ref):
    o_ref[...] = jnp.dot(x_ref[...], w_ref[...],
                         preferred_element_type=jnp.float32)


def _mm(x, w, bm):
    m, k = x.shape
    nc = w.shape[1]
    return pl.pallas_call(
        _mm_body,
        grid=(m // bm,),
        in_specs=[pl.BlockSpec((bm, k), lambda i: (i, 0)),
                  pl.BlockSpec((k, nc), lambda i: (0, 0))],
        out_specs=pl.BlockSpec((bm, nc), lambda i: (i, 0)),
        out_shape=jax.ShapeDtypeStruct((m, nc), jnp.float32),
    )(x, w)


def _emm_body(x_ref, w_ref, o_ref):
    o_ref[0] = jnp.dot(x_ref[...], w_ref[0],
                       preferred_element_type=jnp.float32)


def _edge_mm(ea, w4, bm=1280):
    nl = w4.shape[0]
    nlast = E // bm - 1
    return pl.pallas_call(
        _emm_body,
        grid=(EPAD // bm, nl),
        in_specs=[pl.BlockSpec((bm, ED), lambda i, l: (jnp.minimum(i, nlast), 0)),
                  pl.BlockSpec((1, ED, HD), lambda i, l: (l, 0, 0))],
        out_specs=pl.BlockSpec((1, bm, HD), lambda i, l: (l, i, 0)),
        out_shape=jax.ShapeDtypeStruct((nl, EPAD, HD), jnp.float32),
    )(ea, w4)


def _allsum(v, perms):
    for pidx in perms:
        v = v + v.at[pidx].get(mode="promise_in_bounds")
    return v


def _sc_body(li, dst_hbm, src_hbm, a_hbm, b_hbm, c_hbm, x_hbm,
             w2_hbm, wc_hbm, bc_hbm, outm_hbm, outc_hbm,
             dst_v, src_v, dstd_v, ab, bb, cb, mb, xd, xs,
             d2b, dot2d, rxb, ryb, rzb, cxb, cyb, czb, eix, eiy, eiz, zb,
             w2v, wcv, bcv, accm, accc, sem):
    cid = lax.axis_index("c")
    sid = lax.axis_index("s")
    lane = lax.iota(jnp.int32, LANES)
    perms = [lane ^ k for k in (8, 4, 2, 1)]
    zeros16 = jnp.zeros((LANES,), jnp.float32)

    pltpu.sync_copy(w2_hbm, w2v)
    pltpu.sync_copy(wc_hbm, wcv)
    pltpu.sync_copy(bc_hbm, bcv)

    def _zrow(j, c):
        for r in range(HD // LANES):
            mb[j, pl.ds(r * LANES, LANES)] = zeros16
        return c
    lax.fori_loop(0, KB, _zrow, 0)

    def _zzb(j, c):
        zb[pl.ds(j * LANES, LANES)] = zeros16
        return c
    lax.fori_loop(0, ZBW // LANES, _zzb, 0)
    for t in range(NPH // 16 // KB):
        pltpu.sync_copy(mb, accm.at[pl.ds(sid * (NPH // 16) + t * KB, KB)])
    for t in range(NPCH * HD // 16 // ZBW):
        pltpu.sync_copy(zb, accc.at[pl.ds(sid * (NPCH * HD // 16)
                                          + t * ZBW, ZBW)])
    plsc.subcore_barrier()

    ebase = sid * EPW
    half0 = cid * HALF

    def _blk(b, c):
        base = ebase + b * KB
        pltpu.sync_copy(dst_hbm.at[pl.ds(base, KB)], dst_v)
        pltpu.sync_copy(src_hbm.at[pl.ds(base, KB)], src_v)
        cp_a = pltpu.async_copy(a_hbm.at[dst_v], ab, sem)
        cp_b = pltpu.async_copy(b_hbm.at[src_v], bb, sem)
        cp_xd = pltpu.async_copy(x_hbm.at[dst_v], xd, sem)
        cp_xs = pltpu.async_copy(x_hbm.at[src_v], xs, sem)
        pltpu.sync_copy(c_hbm.at[li, pl.ds(base, KB)], cb)
        cp_a.wait()
        cp_b.wait()
        cp_xd.wait()
        cp_xs.wait()

        for t in range(KB // LANES):
            sl = pl.ds(t * LANES, LANES)
            jvec = lane + t * LANES
            rx = (plsc.load_gather(xd, [jvec, lane * 0])
                  - plsc.load_gather(xs, [jvec, lane * 0]))
            ry = (plsc.load_gather(xd, [jvec, lane * 0 + 1])
                  - plsc.load_gather(xs, [jvec, lane * 0 + 1]))
            rz = (plsc.load_gather(xd, [jvec, lane * 0 + 2])
                  - plsc.load_gather(xs, [jvec, lane * 0 + 2]))
            rxb[sl] = rx
            ryb[sl] = ry
            rzb[sl] = rz
            d2b[sl] = rx * rx + ry * ry + rz * rz
            lo = dst_v[sl] - half0
            own = (lo >= 0) & (lo < HALF)
            dst_v[sl] = jnp.where(own, lo, TRASH)
            dstd_v[sl] = jnp.where(own, lo >> 3, TRASHC)

        def _edge(j, c2):
            d2v = plsc.load_gather(d2b, [lane * 0 + j])
            acc = zeros16
            for r in range(HD // LANES):
                sl = pl.ds(r * LANES, LANES)
                pre = ab[j, sl] + bb[j, sl] + d2v * w2v[sl] + cb[j, sl]
                mv = pre / (1.0 + jnp.exp(-pre))
                mb[j, sl] = mv
                acc = acc + mv * wcv[sl]
            dot2d[j, :] = _allsum(acc, perms)
            return c2
        lax.fori_loop(0, KB, _edge, 0)

        for t in range(KB // LANES):
            sl = pl.ds(t * LANES, LANES)
            jvec = lane + t * LANES
            sv = plsc.load_gather(dot2d, [jvec, lane * 0]) + bcv[pl.ds(0, LANES)]
            ccv = sv / (1.0 + jnp.exp(-sv))
            cxb[sl] = rxb[sl] * ccv
            cyb[sl] = ryb[sl] * ccv
            czb[sl] = rzb[sl] * ccv
            ebase_v = dstd_v[sl] * HD + (dst_v[sl] & 7) * LANES
            eix[sl] = ebase_v
            eiy[sl] = ebase_v + 1
            eiz[sl] = ebase_v + 2

        pltpu.sync_copy(mb, accm.at[dst_v], add=True)
        pltpu.sync_copy(cxb, accc.at[eix], add=True)
        pltpu.sync_copy(cyb, accc.at[eiy], add=True)
        pltpu.sync_copy(czb, accc.at[eiz], add=True)
        return c
    lax.fori_loop(0, NBLK, _blk, 0)
    plsc.subcore_barrier()

    mrow0 = sid * MROWS_PT
    for t in range(MROWS_PT // KB):
        pltpu.sync_copy(accm.at[pl.ds(mrow0 + t * KB, KB)],
                        outm_hbm.at[pl.ds(half0 + mrow0 + t * KB, KB)])
    cw0 = sid * CROWS_PT * HD
    pltpu.sync_copy(accc.at[pl.ds(cw0, CROWS_PT * HD)],
                    outc_hbm.at[pl.ds(cid * CHALF * HD + cw0, CROWS_PT * HD)])


@functools.cache
def _make_sc(li):
    mesh = plsc.VectorSubcoreMesh(core_axis_name="c", subcore_axis_name="s")
    return pl.kernel(
        functools.partial(_sc_body, li),
        out_type=[jax.ShapeDtypeStruct((NP, HD), jnp.float32),
                  jax.ShapeDtypeStruct((NPC * HD,), jnp.float32)],
        mesh=mesh,
        compiler_params=pltpu.CompilerParams(needs_layout_passes=False),
        scratch_types=[
            pltpu.VMEM((KB,), jnp.int32),
            pltpu.VMEM((KB,), jnp.int32),
            pltpu.VMEM((KB,), jnp.int32),
            pltpu.VMEM((KB, HD), jnp.float32),
            pltpu.VMEM((KB, HD), jnp.float32),
            pltpu.VMEM((KB, HD), jnp.float32),
            pltpu.VMEM((KB, HD), jnp.float32),
            pltpu.VMEM((KB, HD), jnp.float32),
            pltpu.VMEM((KB, HD), jnp.float32),
            pltpu.VMEM((KB,), jnp.float32),
            pltpu.VMEM((KB, LANES), jnp.float32),
            pltpu.VMEM((KB,), jnp.float32),
            pltpu.VMEM((KB,), jnp.float32),
            pltpu.VMEM((KB,), jnp.float32),
            pltpu.VMEM((KB,), jnp.float32),
            pltpu.VMEM((KB,), jnp.float32),
            pltpu.VMEM((KB,), jnp.float32),
            pltpu.VMEM((KB,), jnp.int32),
            pltpu.VMEM((KB,), jnp.int32),
            pltpu.VMEM((KB,), jnp.int32),
            pltpu.VMEM((ZBW,), jnp.float32),
            pltpu.VMEM((HD,), jnp.float32),
            pltpu.VMEM((HD,), jnp.float32),
            pltpu.VMEM((HD,), jnp.float32),
            pltpu.VMEM_SHARED((NPH, HD), jnp.float32),
            pltpu.VMEM_SHARED((NPCH * HD,), jnp.float32),
            pltpu.SemaphoreType.DMA,
        ],
    )


def kernel(h, pos, edge_index, edge_attr, params):
    src = edge_index[0].astype(jnp.int32)
    dst = edge_index[1].astype(jnp.int32)
    pad_idx = jnp.full((EPAD - E,), DUMMY, jnp.int32)
    srcp = jnp.concatenate([src, pad_idx])
    dstp = jnp.concatenate([dst, pad_idx])
    layers = [params[k] for k in ("l1", "l2", "l3", "l4")]

    wa4 = jnp.stack([p["We"][2 * HD + 1:] for p in layers])
    cedge = _edge_mm(edge_attr, wa4)

    x = pos
    hcur = h
    for i, p in enumerate(layers):
        wab = jnp.concatenate([p["We"][:HD], p["We"][HD:2 * HD]], axis=1)
        ab_tab = _mm(hcur, wab, 1000)
        a_tab = jnp.zeros((NP, HD), jnp.float32).at[:N].set(
            ab_tab[:, :HD] + p["be"])
        b_tab = jnp.zeros((NP, HD), jnp.float32).at[:N].set(ab_tab[:, HD:])
        xpad = jnp.zeros((NP, HD), jnp.float32).at[:N, :3].set(x)
        w2 = p["We"][2 * HD]
        wc = p["Wc"][:, 0]
        bc = jnp.full((HD,), p["bc"][0], jnp.float32)
        pm, pc = _make_sc(i)(dstp, srcp, a_tab, b_tab, cedge, xpad,
                             w2, wc, bc)
        agg_m = pm[:N]
        agg_c = pc.reshape(NPC, 8, LANES)[:, :, :3].reshape(NP, 3)[:N]
        x = x + p["cs"] * agg_c
        hn = _mm(jnp.concatenate([hcur, agg_m], axis=1), p["Wn"], 1000)
        hn = hn + p["bn"]
        hn = hn * jax.nn.sigmoid(hn)
        mu = jnp.mean(hn, axis=0)
        var = jnp.var(hn, axis=0)
        hb = p["g"] * (hn - mu) / jnp.sqrt(var + 1e-5) + p["b"]
        hcur = hb * jax.nn.sigmoid(hb) if i < 3 else hb
    return hcur
